# baseline (device time: 7342 ns/iter reference)
import jax
import jax.numpy as jnp
from jax import lax
from jax.experimental import pallas as pl
from jax.experimental.pallas import tpu as pltpu

N_DEV = 4


def kernel(x):
    m_per, n = x.shape

    def body(x_ref, out_ref, total_ref, recv_buf, send_sems, recv_sems):
        my_pos = lax.axis_index("i")

        barrier_sem = pltpu.get_barrier_semaphore()
        for p in range(1, N_DEV):
            @pl.when(my_pos >= p)
            def _(p=p):
                pl.semaphore_signal(
                    barrier_sem, inc=1,
                    device_id=((my_pos - p) % N_DEV,),
                    device_id_type=pl.DeviceIdType.MESH,
                )

        x = x_ref[:, :]
        v = x
        k = 1
        while k < 256:
            shifted = jnp.concatenate(
                [jnp.ones((k, n), v.dtype), v[: m_per - k, :]], axis=0
            )
            v = v * shifted
            k *= 2

        total_ref[:, :] = (v[255:256, :] * v[511:512, :]) * (
            v[767:768, :] * v[1023:1024, :]
        )

        for c in range(N_DEV - 1):
            @pl.when(my_pos == c)
            def _(c=c):
                pl.semaphore_wait(barrier_sem, N_DEV - 1 - c)

        for d in range(1, N_DEV):
            @pl.when(my_pos + d < N_DEV)
            def _(d=d):
                pltpu.make_async_remote_copy(
                    src_ref=total_ref,
                    dst_ref=recv_buf.at[d],
                    send_sem=send_sems.at[d],
                    recv_sem=recv_sems.at[d],
                    device_id=((my_pos + d) % N_DEV,),
                    device_id_type=pl.DeviceIdType.MESH,
                ).start()

        while k < m_per:
            shifted = jnp.concatenate(
                [jnp.ones((k, n), v.dtype), v[: m_per - k, :]], axis=0
            )
            v = v * shifted
            k *= 2

        for d in range(1, N_DEV):
            @pl.when(my_pos >= d)
            def _(d=d):
                pltpu.make_async_remote_copy(
                    src_ref=total_ref,
                    dst_ref=recv_buf.at[d],
                    send_sem=send_sems.at[d],
                    recv_sem=recv_sems.at[d],
                    device_id=((my_pos - d) % N_DEV,),
                    device_id_type=pl.DeviceIdType.MESH,
                ).wait_recv()

        prefix = jnp.ones((1, n), jnp.float32)
        for d in range(1, N_DEV):
            prefix = prefix * jnp.where(my_pos >= d, recv_buf[d], 1.0)
        out_ref[:, :] = v * prefix

        for d in range(1, N_DEV):
            @pl.when(my_pos + d < N_DEV)
            def _(d=d):
                pltpu.make_async_remote_copy(
                    src_ref=total_ref,
                    dst_ref=recv_buf.at[d],
                    send_sem=send_sems.at[d],
                    recv_sem=recv_sems.at[d],
                    device_id=((my_pos + d) % N_DEV,),
                    device_id_type=pl.DeviceIdType.MESH,
                ).wait_send()

    return pl.pallas_call(
        body,
        out_shape=jax.ShapeDtypeStruct((m_per, n), x.dtype),
        in_specs=[pl.BlockSpec(memory_space=pltpu.VMEM)],
        out_specs=pl.BlockSpec(memory_space=pltpu.VMEM),
        scratch_shapes=[
            pltpu.VMEM((1, n), x.dtype),
            pltpu.VMEM((N_DEV, 1, n), x.dtype),
            pltpu.SemaphoreType.DMA((N_DEV,)),
            pltpu.SemaphoreType.DMA((N_DEV,)),
        ],
        compiler_params=pltpu.CompilerParams(collective_id=0),
    )(x)
